# baseline (speedup 1.0000x reference)
import functools
import math

import jax
import jax.numpy as jnp
from jax import lax
from jax.experimental import pallas as pl
from jax.experimental.pallas import tpu as pltpu
from jax.experimental.pallas import tpu_sc as plsc

N = 100000
E = 1600000
D = 128
H = 128
BN_SCALE = 1.0 / math.sqrt(1.0 + 1e-3)

FH = 64
R2 = 25600
NR2 = 4
TPS = R2 // 16
C = 512
NCHT = E // C
DUMMY = R2

BLK = 2000


def _prep_body(x_ref, s1_ref, t1_ref, w1_ref, b1_ref, s2_ref, t2_ref,
               w2_ref, b2_ref, lo_ref, hi_ref):
    h = x_ref[...] * s1_ref[...] + t1_ref[...]
    h = jax.nn.gelu(jnp.dot(h, w1_ref[...], preferred_element_type=jnp.float32)
                    + b1_ref[...])
    h = h * s2_ref[...] + t2_ref[...]
    h = jax.nn.gelu(
        jnp.dot(h, w2_ref[...], preferred_element_type=jnp.float32) + b2_ref[...])
    lo_ref[...] = h[:, :FH]
    hi_ref[...] = h[:, FH:]


def _row_spec(rows, cols):
    return pl.BlockSpec((rows, cols), lambda i: (i, 0))


def _full_spec(shape):
    return pl.BlockSpec(shape, lambda i: (0,) * len(shape))


def _prep_ffn(x, s1, t1, w1, b1, s2, t2, w2, b2):
    grid = (N // BLK,)
    return pl.pallas_call(
        _prep_body,
        grid=grid,
        in_specs=[
            _row_spec(BLK, D),
            _full_spec((1, D)), _full_spec((1, D)),
            _full_spec((D, H)), _full_spec((1, H)),
            _full_spec((1, H)), _full_spec((1, H)),
            _full_spec((H, H)), _full_spec((1, H)),
        ],
        out_specs=[_row_spec(BLK, FH), _row_spec(BLK, FH)],
        out_shape=[jax.ShapeDtypeStruct((N, FH), jnp.float32),
                   jax.ShapeDtypeStruct((N, FH), jnp.float32)],
    )(x, s1, t1, w1, b1, s2, t2, w2, b2)


def _upd_body(x_ref, alo_ref, ahi_ref, s1x_ref, t1x_ref, s1lo_ref, t1lo_ref,
              s1hi_ref, t1hi_ref, w1x_ref, w1lo_ref, w1hi_ref, b1_ref,
              s2_ref, t2_ref, w2_ref, b2_ref, o_ref):
    xs = x_ref[...] * s1x_ref[...] + t1x_ref[...]
    alo = alo_ref[...] * s1lo_ref[...] + t1lo_ref[...]
    ahi = ahi_ref[...] * s1hi_ref[...] + t1hi_ref[...]
    h = (jnp.dot(xs, w1x_ref[...], preferred_element_type=jnp.float32)
         + jnp.dot(alo, w1lo_ref[...], preferred_element_type=jnp.float32)
         + jnp.dot(ahi, w1hi_ref[...], preferred_element_type=jnp.float32)
         + b1_ref[...])
    h = jax.nn.gelu(h)
    h = h * s2_ref[...] + t2_ref[...]
    h = jax.nn.gelu(jnp.dot(h, w2_ref[...], preferred_element_type=jnp.float32)
                    + b2_ref[...])
    norm = jnp.sqrt(jnp.sum(h * h, axis=-1, keepdims=True))
    o_ref[...] = h / jnp.maximum(norm, 1e-12)


def _upd_ffn(x, agg_lo, agg_hi, s1x, t1x, s1lo, t1lo, s1hi, t1hi,
             w1x, w1lo, w1hi, b1, s2, t2, w2, b2):
    grid = (N // BLK,)
    return pl.pallas_call(
        _upd_body,
        grid=grid,
        in_specs=[
            _row_spec(BLK, D),
            _row_spec(BLK, FH),
            _row_spec(BLK, FH),
            _full_spec((1, D)), _full_spec((1, D)),
            _full_spec((1, FH)), _full_spec((1, FH)),
            _full_spec((1, FH)), _full_spec((1, FH)),
            _full_spec((D, H)), _full_spec((FH, H)), _full_spec((FH, H)),
            _full_spec((1, H)),
            _full_spec((1, H)), _full_spec((1, H)),
            _full_spec((H, H)), _full_spec((1, H)),
        ],
        out_specs=_row_spec(BLK, H),
        out_shape=jax.ShapeDtypeStruct((N, H), jnp.float32),
    )(x, agg_lo, agg_hi, s1x, t1x, s1lo, t1lo, s1hi, t1hi,
      w1x, w1lo, w1hi, b1, s2, t2, w2, b2)


def _edge_agg(tab, dst_arr, src_arr, ew):
    mesh = plsc.VectorSubcoreMesh(core_axis_name="c", subcore_axis_name="s")

    @functools.partial(
        pl.kernel,
        out_type=jax.ShapeDtypeStruct((NR2 * R2, FH), jnp.float32),
        mesh=mesh,
        compiler_params=pltpu.CompilerParams(use_tc_tiling_on_sc=False),
        scratch_types=[
            pltpu.VMEM((C,), jnp.int32),
            pltpu.VMEM((C,), jnp.int32),
            pltpu.VMEM((C,), jnp.float32),
            pltpu.VMEM((C,), jnp.int32),
            pltpu.VMEM((C,), jnp.int32),
            pltpu.VMEM((128, FH), jnp.float32),
            pltpu.VMEM((160, FH), jnp.float32),
            pltpu.VMEM_SHARED((R2 + 16, FH), jnp.float32),
            pltpu.SemaphoreType.DMA,
        ],
    )
    def k(tab_hbm, dst_hbm, src_hbm, ew_hbm, out_hbm,
          dst_c, src_c, w_c, gidx, dloc, rows, zeros, acc, sem):
        cid = lax.axis_index("c")
        sid = lax.axis_index("s")
        zero16f = jnp.zeros((16,), jnp.float32)

        def zinit(j, _):
            def zf(f, __):
                zeros[j, pl.ds(f * 16, 16)] = zero16f
                return 0
            return lax.fori_loop(0, FH // 16, zf, 0)
        lax.fori_loop(0, 160, zinit, 0)

        nch = 195 + (jnp.right_shift(sid - 5, 31) & 1)

        for rr in range(NR2 // 2):
            rid = rr * 2 + cid
            lo = rid * R2
            for kk in range(10):
                pltpu.sync_copy(zeros, acc.at[pl.ds(sid * TPS + kk * 160, 160)])
            plsc.subcore_barrier()

            def chunk_body(ch, _):
                base = (ch * 16 + sid) * C
                pltpu.sync_copy(dst_hbm.at[pl.ds(base, C)], dst_c)
                pltpu.sync_copy(src_hbm.at[pl.ds(base, C)], src_c)
                pltpu.sync_copy(ew_hbm.at[pl.ds(base, C)], w_c)

                def vf(i, __):
                    d = dst_c[pl.ds(i * 16, 16)]
                    s = src_c[pl.ds(i * 16, 16)]
                    dl = d - lo
                    hit = (jnp.right_shift(dl - R2, 31)
                           & ~jnp.right_shift(dl, 31) & 1)
                    gidx[pl.ds(i * 16, 16)] = hit * s
                    dloc[pl.ds(i * 16, 16)] = hit * dl + (1 - hit) * DUMMY
                    return 0
                lax.fori_loop(0, C // 16, vf, 0)

                def drain(t, __):
                    tb = t * 128
                    pltpu.async_copy(
                        tab_hbm.at[gidx.at[pl.ds(tb, 128)]], rows, sem
                    ).wait()

                    def group(g, ___):
                        gb = tb + g * 16
                        dl16 = dloc[pl.ds(gb, 16)]
                        w16 = w_c[pl.ds(gb, 16)]
                        for l in range(16):
                            ws = w16[l]
                            r0 = g * 16 + l
                            for f in range(FH // 16):
                                rows[r0, pl.ds(f * 16, 16)] = (
                                    rows[r0, pl.ds(f * 16, 16)] * ws)
                        pltpu.sync_copy(rows.at[pl.ds(g * 16, 16)],
                                        acc.at[dl16], add=True)
                        return 0
                    lax.fori_loop(0, 8, group, 0)
                    return 0
                lax.fori_loop(0, C // 128, drain, 0)
                return 0
            lax.fori_loop(0, nch, chunk_body, 0)
            plsc.subcore_barrier()

            for kk in range(10):
                off = sid * TPS + kk * 160
                pltpu.sync_copy(acc.at[pl.ds(off, 160)],
                                out_hbm.at[pl.ds(lo + off, 160)])
            plsc.subcore_barrier()

    return k(tab, dst_arr, src_arr, ew)


def kernel(node_representations, edges, edge_weights,
           prep_bn1_gamma, prep_bn1_beta, prep_dense1_W, prep_dense1_b,
           prep_bn2_gamma, prep_bn2_beta, prep_dense2_W, prep_dense2_b,
           upd_bn1_gamma, upd_bn1_beta, upd_dense1_W, upd_dense1_b,
           upd_bn2_gamma, upd_bn2_beta, upd_dense2_W, upd_dense2_b):
    f32 = jnp.float32

    prep_lo, prep_hi = _prep_ffn(
        node_representations,
        (prep_bn1_gamma * BN_SCALE)[None, :].astype(f32),
        prep_bn1_beta[None, :],
        prep_dense1_W, prep_dense1_b[None, :],
        (prep_bn2_gamma * BN_SCALE)[None, :].astype(f32),
        prep_bn2_beta[None, :],
        prep_dense2_W, prep_dense2_b[None, :],
    )

    dst_arr = edges[0]
    src_arr = edges[1]
    agg_lo = _edge_agg(prep_lo, dst_arr, src_arr, edge_weights)
    agg_hi = _edge_agg(prep_hi, dst_arr, src_arr, edge_weights)

    out = _upd_ffn(
        node_representations, agg_lo, agg_hi,
        (upd_bn1_gamma[:D] * BN_SCALE)[None, :].astype(f32),
        upd_bn1_beta[None, :D],
        (upd_bn1_gamma[D:D + FH] * BN_SCALE)[None, :].astype(f32),
        upd_bn1_beta[None, D:D + FH],
        (upd_bn1_gamma[D + FH:] * BN_SCALE)[None, :].astype(f32),
        upd_bn1_beta[None, D + FH:],
        upd_dense1_W[:D], upd_dense1_W[D:D + FH], upd_dense1_W[D + FH:],
        upd_dense1_b[None, :],
        (upd_bn2_gamma * BN_SCALE)[None, :].astype(f32),
        upd_bn2_beta[None, :],
        upd_dense2_W, upd_dense2_b[None, :],
    )
    return out
